# baseline (speedup 1.0000x reference)
import jax
import jax.numpy as jnp
from jax import lax
from jax.experimental import pallas as pl
from jax.experimental.pallas import tpu as pltpu
from jax.experimental.pallas import tpu_sc as plsc

S, D, F, E = 2048, 768, 3072, 8
DP = D // 2
T = 512
NBLK = 16
NSLOT = NBLK * T
JF = 4
Fb = F // JF
SB = 8
RB = S // SB
NC, NS = 2, 16
NW = NC * NS
TOK_W = S // NW


def _pack_bf16_pairs(x32):
    xu = lax.bitcast_convert_type(x32.astype(jnp.bfloat16), jnp.uint16)
    lo = xu[:, :DP].astype(jnp.uint32)
    hi = xu[:, DP:].astype(jnp.uint32)
    return lax.bitcast_convert_type(lo | (hi << 16), jnp.int32)


def _unpack_bf16_pairs(xi):
    xu = lax.bitcast_convert_type(xi, jnp.uint32)
    lo = (xu & 0xFFFF).astype(jnp.uint16)
    hi = (xu >> 16).astype(jnp.uint16)
    return jnp.concatenate(
        [lax.bitcast_convert_type(lo, jnp.bfloat16),
         lax.bitcast_convert_type(hi, jnp.bfloat16)], axis=1)


def _router_body(x_ref, gw_ref, logits_ref, w1_ref, w2_ref, xpk_ref,
                 slot1_ref, slot2_ref, be_ref, nt_ref,
                 cnt_acc, posin_s, i1_s, i2_s):
    i = pl.program_id(0)

    @pl.when(i == 0)
    def _init():
        cnt_acc[...] = jnp.zeros_like(cnt_acc)

    x = x_ref[...]
    logits = jnp.dot(x, gw_ref[...],
                     preferred_element_type=jnp.float32)
    iota = lax.broadcasted_iota(jnp.int32, (RB, E), 1)
    m1 = jnp.max(logits, axis=1, keepdims=True)
    idx1 = jnp.min(jnp.where(logits == m1, iota, E), axis=1, keepdims=True)
    oh1 = iota == idx1
    masked = jnp.where(oh1, -jnp.inf, logits)
    m2 = jnp.max(masked, axis=1, keepdims=True)
    idx2 = jnp.min(jnp.where(masked == m2, iota, E), axis=1, keepdims=True)
    oh2 = iota == idx2
    w1 = 1.0 / (1.0 + jnp.exp(m2 - m1))

    mask = (oh1 | oh2).astype(jnp.float32)
    r = lax.broadcasted_iota(jnp.int32, (RB, RB), 0)
    c = lax.broadcasted_iota(jnp.int32, (RB, RB), 1)
    tril = (r > c).astype(jnp.bfloat16)
    posin = jnp.dot(tril, mask.astype(jnp.bfloat16),
                    preferred_element_type=jnp.float32) + cnt_acc[...]

    logits_ref[...] = logits
    w1_ref[...] = jnp.broadcast_to(w1, (RB, 16))
    w2_ref[...] = jnp.broadcast_to(1.0 - w1, (RB, 16))
    xpk_ref[...] = _pack_bf16_pairs(x)
    posin_s[pl.ds(i * RB, RB), :] = posin
    i1_s[pl.ds(i * RB, RB), :] = idx1
    i2_s[pl.ds(i * RB, RB), :] = idx2
    cnt_acc[...] += jnp.sum(mask, axis=0, keepdims=True)

    @pl.when(i == SB - 1)
    def _fin():
        counts = cnt_acc[...]
        nb = jnp.ceil(counts / T)
        lane8 = lax.broadcasted_iota(jnp.int32, (1, E), 1)
        baseb = jnp.zeros_like(nb)
        for e in range(E - 1):
            baseb += jnp.where(lane8 > e, nb[:, e:e + 1], 0.0)
        cum_incl = baseb + nb
        ntot = jnp.sum(nb, axis=1, keepdims=True)

        slotmat = baseb * T + posin_s[...]
        iota_se = lax.broadcasted_iota(jnp.int32, (S, E), 1)
        o1 = iota_se == i1_s[...]
        o2 = iota_se == i2_s[...]
        slot1_ref[...] = jnp.sum(jnp.where(o1, slotmat, 0.0),
                                 axis=1).astype(jnp.int32)
        slot2_ref[...] = jnp.sum(jnp.where(o2, slotmat, 0.0),
                                 axis=1).astype(jnp.int32)

        ib = lax.broadcasted_iota(jnp.int32, (1, NBLK), 1).astype(jnp.float32)
        raw = jnp.zeros((1, NBLK), jnp.float32)
        el = jnp.zeros((1, 1), jnp.float32)
        for e in range(E):
            raw += (cum_incl[:, e:e + 1] <= ib).astype(jnp.float32)
            el += (cum_incl[:, e:e + 1] <= ntot - 1.0).astype(jnp.float32)
        be_ref[...] = jnp.minimum(raw, el).astype(jnp.int32).reshape(NBLK)
        nt_ref[...] = ntot.astype(jnp.int32).reshape(1)


def _dispatch_body(xpk_hbm, s1_hbm, s2_hbm, xs_hbm, xb, i1, i2, sem):
    wid = lax.axis_index("s") * NC + lax.axis_index("c")
    base = wid * TOK_W
    l1 = pltpu.async_copy(s1_hbm.at[pl.ds(base, TOK_W)], i1, sem)
    l2 = pltpu.async_copy(s2_hbm.at[pl.ds(base, TOK_W)], i2, sem)
    l3 = pltpu.async_copy(xpk_hbm.at[pl.ds(base, TOK_W)], xb, sem)
    l1.wait()
    l2.wait()
    l3.wait()
    c1 = pltpu.async_copy(xb, xs_hbm.at[i1], sem)
    c2 = pltpu.async_copy(xb, xs_hbm.at[i2], sem)
    c1.wait()
    c2.wait()


def _mlp_body(be_ref, nt_ref, xs_ref, wi_ref, wv_ref, wo_ref, ys_ref, acc):
    j = pl.program_id(0)
    i = pl.program_id(1)

    @pl.when(i < nt_ref[0])
    def _compute():
        xb = _unpack_bf16_pairs(xs_ref[...])
        a = jnp.dot(xb, wi_ref[0].astype(jnp.bfloat16),
                    preferred_element_type=jnp.float32)
        b = jnp.dot(xb, wv_ref[0].astype(jnp.bfloat16),
                    preferred_element_type=jnp.float32)
        c1 = 0.7978845608028654
        c2 = 0.7978845608028654 * 0.044715
        t = jnp.tanh(a * (c1 + c2 * (a * a)))
        half_a = 0.5 * a
        h = ((half_a + half_a * t) * b).astype(jnp.bfloat16)
        y = jnp.dot(h, wo_ref[0].astype(jnp.bfloat16),
                    preferred_element_type=jnp.float32)

        @pl.when(j == 0)
        def _set():
            acc[pl.ds(i * T, T), :] = y

        @pl.when((j > 0) & (j < JF - 1))
        def _add():
            acc[pl.ds(i * T, T), :] += y

        @pl.when(j == JF - 1)
        def _fin():
            ys_ref[...] = _pack_bf16_pairs(acc[pl.ds(i * T, T), :] + y)


def _combine_body(ys_hbm, s1_hbm, s2_hbm, w1_hbm, w2_hbm, out_hbm,
                  i1, i2, wv1, wv2, b1, b2, bout, sem):
    wid = lax.axis_index("s") * NC + lax.axis_index("c")
    base = wid * TOK_W
    l1 = pltpu.async_copy(s1_hbm.at[pl.ds(base, TOK_W)], i1, sem)
    l2 = pltpu.async_copy(s2_hbm.at[pl.ds(base, TOK_W)], i2, sem)
    l3 = pltpu.async_copy(w1_hbm.at[pl.ds(base, TOK_W)], wv1, sem)
    l4 = pltpu.async_copy(w2_hbm.at[pl.ds(base, TOK_W)], wv2, sem)
    l1.wait()
    l2.wait()
    l3.wait()
    l4.wait()
    c1 = pltpu.async_copy(ys_hbm.at[i1], b1, sem)
    c2 = pltpu.async_copy(ys_hbm.at[i2], b2, sem)
    c1.wait()
    c2.wait()

    def row(t, _):
        w1s = wv1[t]
        w2s = wv2[t]

        def col(c, _):
            v1 = b1[t, pl.ds(c * 16, 16)]
            v2 = b2[t, pl.ds(c * 16, 16)]
            lo1 = lax.bitcast_convert_type(v1 << 16, jnp.float32)
            lo2 = lax.bitcast_convert_type(v2 << 16, jnp.float32)
            hi1 = lax.bitcast_convert_type(v1 & jnp.int32(-65536), jnp.float32)
            hi2 = lax.bitcast_convert_type(v2 & jnp.int32(-65536), jnp.float32)
            bout[t, pl.ds(c * 16, 16)] = lo1 * w1s + lo2 * w2s
            bout[t, pl.ds(DP + c * 16, 16)] = hi1 * w1s + hi2 * w2s
            return 0
        return lax.fori_loop(0, DP // 16, col, 0, unroll=8)

    lax.fori_loop(0, TOK_W, row, 0)
    pltpu.sync_copy(bout, out_hbm.at[pl.ds(base, TOK_W)])


def kernel(hidden_states, gate_w, w_in, w_v, w_out):
    B = hidden_states.shape[0]
    x = hidden_states.reshape(S, D)

    logits, w1, w2, xpk, slot1, slot2, be, nt = pl.pallas_call(
        _router_body,
        grid=(SB,),
        in_specs=[
            pl.BlockSpec((RB, D), lambda i: (i, 0)),
            pl.BlockSpec((D, E), lambda i: (0, 0)),
        ],
        out_specs=[
            pl.BlockSpec((RB, E), lambda i: (i, 0)),
            pl.BlockSpec((RB, 16), lambda i: (i, 0)),
            pl.BlockSpec((RB, 16), lambda i: (i, 0)),
            pl.BlockSpec((RB, DP), lambda i: (i, 0)),
            pl.BlockSpec((S,), lambda i: (0,)),
            pl.BlockSpec((S,), lambda i: (0,)),
            pl.BlockSpec((NBLK,), lambda i: (0,)),
            pl.BlockSpec((1,), lambda i: (0,)),
        ],
        out_shape=(
            jax.ShapeDtypeStruct((S, E), jnp.float32),
            jax.ShapeDtypeStruct((S, 16), jnp.float32),
            jax.ShapeDtypeStruct((S, 16), jnp.float32),
            jax.ShapeDtypeStruct((S, DP), jnp.int32),
            jax.ShapeDtypeStruct((S,), jnp.int32),
            jax.ShapeDtypeStruct((S,), jnp.int32),
            jax.ShapeDtypeStruct((NBLK,), jnp.int32),
            jax.ShapeDtypeStruct((1,), jnp.int32),
        ),
        scratch_shapes=[
            pltpu.VMEM((1, E), jnp.float32),
            pltpu.VMEM((S, E), jnp.float32),
            pltpu.VMEM((S, 1), jnp.int32),
            pltpu.VMEM((S, 1), jnp.int32),
        ],
    )(x, gate_w)

    mesh = plsc.VectorSubcoreMesh(core_axis_name="c", subcore_axis_name="s")
    xs = pl.kernel(
        _dispatch_body,
        out_type=jax.ShapeDtypeStruct((NSLOT, DP), jnp.int32),
        mesh=mesh,
        scratch_types=[
            pltpu.VMEM((TOK_W, DP), jnp.int32),
            pltpu.VMEM((TOK_W,), jnp.int32),
            pltpu.VMEM((TOK_W,), jnp.int32),
            pltpu.SemaphoreType.DMA,
        ],
    )(xpk, slot1, slot2)

    ys = pl.pallas_call(
        _mlp_body,
        grid_spec=pltpu.PrefetchScalarGridSpec(
            num_scalar_prefetch=2,
            grid=(JF, NBLK),
            in_specs=[
                pl.BlockSpec(
                    (T, DP), lambda j, i, be, nt: (jnp.minimum(i, nt[0] - 1), 0)),
                pl.BlockSpec(
                    (1, D, Fb), lambda j, i, be, nt: (be[i], 0, j)),
                pl.BlockSpec(
                    (1, D, Fb), lambda j, i, be, nt: (be[i], 0, j)),
                pl.BlockSpec(
                    (1, Fb, D), lambda j, i, be, nt: (be[i], j, 0)),
            ],
            out_specs=pl.BlockSpec(
                (T, DP),
                lambda j, i, be, nt: (
                    jnp.where(j == 0, 0, jnp.minimum(i, nt[0] - 1)), 0)),
            scratch_shapes=[pltpu.VMEM((NSLOT, D), jnp.float32)],
        ),
        out_shape=jax.ShapeDtypeStruct((NSLOT, DP), jnp.int32),
    )(be, nt, xs, w_in, w_v, w_out)

    out = pl.kernel(
        _combine_body,
        out_type=jax.ShapeDtypeStruct((S, D), jnp.float32),
        mesh=mesh,
        scratch_types=[
            pltpu.VMEM((TOK_W,), jnp.int32),
            pltpu.VMEM((TOK_W,), jnp.int32),
            pltpu.VMEM((TOK_W, 16), jnp.float32),
            pltpu.VMEM((TOK_W, 16), jnp.float32),
            pltpu.VMEM((TOK_W, DP), jnp.int32),
            pltpu.VMEM((TOK_W, DP), jnp.int32),
            pltpu.VMEM((TOK_W, D), jnp.float32),
            pltpu.SemaphoreType.DMA,
        ],
    )(ys, slot1, slot2, w1, w2)

    return out.reshape(B, S, D), logits.reshape(B, S, E)


# speedup vs baseline: 1.1563x; 1.1563x over previous
import jax
import jax.numpy as jnp
from jax import lax
from jax.experimental import pallas as pl
from jax.experimental.pallas import tpu as pltpu
from jax.experimental.pallas import tpu_sc as plsc

S, D, F, E = 2048, 768, 3072, 8
DP = D // 2
T = 512
NBLK = 16
NSLOT = NBLK * T
JF = 2
Fb = F // JF
SB = 4
RB = S // SB
NC, NS = 2, 16
NW = NC * NS
TOK_W = S // NW


def _pack_bf16_pairs(x32):
    xu = lax.bitcast_convert_type(x32.astype(jnp.bfloat16), jnp.uint16)
    lo = xu[:, :DP].astype(jnp.uint32)
    hi = xu[:, DP:].astype(jnp.uint32)
    return lax.bitcast_convert_type(lo | (hi << 16), jnp.int32)


def _unpack_bf16_pairs(xi):
    xu = lax.bitcast_convert_type(xi, jnp.uint32)
    lo = (xu & 0xFFFF).astype(jnp.uint16)
    hi = (xu >> 16).astype(jnp.uint16)
    return jnp.concatenate(
        [lax.bitcast_convert_type(lo, jnp.bfloat16),
         lax.bitcast_convert_type(hi, jnp.bfloat16)], axis=1)


def _router_body(x_ref, gw_ref, logits_ref, w1_ref, w2_ref, xpk_ref,
                 slot1_ref, slot2_ref, be_ref, nt_ref,
                 cnt_acc, posin_s, i1_s, i2_s):
    i = pl.program_id(0)

    @pl.when(i == 0)
    def _init():
        cnt_acc[...] = jnp.zeros_like(cnt_acc)

    x = x_ref[...]
    logits = jnp.dot(x, gw_ref[...],
                     preferred_element_type=jnp.float32)
    iota = lax.broadcasted_iota(jnp.int32, (RB, E), 1)
    m1 = jnp.max(logits, axis=1, keepdims=True)
    idx1 = jnp.min(jnp.where(logits == m1, iota, E), axis=1, keepdims=True)
    oh1 = iota == idx1
    masked = jnp.where(oh1, -jnp.inf, logits)
    m2 = jnp.max(masked, axis=1, keepdims=True)
    idx2 = jnp.min(jnp.where(masked == m2, iota, E), axis=1, keepdims=True)
    oh2 = iota == idx2
    w1 = 1.0 / (1.0 + jnp.exp(m2 - m1))

    mask = (oh1 | oh2).astype(jnp.float32)
    r = lax.broadcasted_iota(jnp.int32, (RB, RB), 0)
    c = lax.broadcasted_iota(jnp.int32, (RB, RB), 1)
    tril = (r > c).astype(jnp.bfloat16)
    posin = jnp.dot(tril, mask.astype(jnp.bfloat16),
                    preferred_element_type=jnp.float32) + cnt_acc[...]

    logits_ref[...] = logits
    w1_ref[...] = jnp.broadcast_to(w1, (RB, 16))
    w2_ref[...] = jnp.broadcast_to(1.0 - w1, (RB, 16))
    xpk_ref[...] = _pack_bf16_pairs(x)
    posin_s[pl.ds(i * RB, RB), :] = posin
    i1_s[pl.ds(i * RB, RB), :] = idx1
    i2_s[pl.ds(i * RB, RB), :] = idx2
    cnt_acc[...] += jnp.sum(mask, axis=0, keepdims=True)

    @pl.when(i == SB - 1)
    def _fin():
        counts = cnt_acc[...]
        nb = jnp.ceil(counts / T)
        lane8 = lax.broadcasted_iota(jnp.int32, (1, E), 1)
        baseb = jnp.zeros_like(nb)
        for e in range(E - 1):
            baseb += jnp.where(lane8 > e, nb[:, e:e + 1], 0.0)
        cum_incl = baseb + nb
        ntot = jnp.sum(nb, axis=1, keepdims=True)

        slotmat = baseb * T + posin_s[...]
        iota_se = lax.broadcasted_iota(jnp.int32, (S, E), 1)
        o1 = iota_se == i1_s[...]
        o2 = iota_se == i2_s[...]
        slot1_ref[...] = jnp.sum(jnp.where(o1, slotmat, 0.0),
                                 axis=1).astype(jnp.int32)
        slot2_ref[...] = jnp.sum(jnp.where(o2, slotmat, 0.0),
                                 axis=1).astype(jnp.int32)

        ib = lax.broadcasted_iota(jnp.int32, (1, NBLK), 1).astype(jnp.float32)
        raw = jnp.zeros((1, NBLK), jnp.float32)
        el = jnp.zeros((1, 1), jnp.float32)
        for e in range(E):
            raw += (cum_incl[:, e:e + 1] <= ib).astype(jnp.float32)
            el += (cum_incl[:, e:e + 1] <= ntot - 1.0).astype(jnp.float32)
        be_ref[...] = jnp.minimum(raw, el).astype(jnp.int32).reshape(NBLK)
        nt_ref[...] = ntot.astype(jnp.int32).reshape(1)


def _dispatch_body(xpk_hbm, s1_hbm, s2_hbm, xs_hbm, xb, i1, i2, sem):
    wid = lax.axis_index("s") * NC + lax.axis_index("c")
    base = wid * TOK_W
    l1 = pltpu.async_copy(s1_hbm.at[pl.ds(base, TOK_W)], i1, sem)
    l2 = pltpu.async_copy(s2_hbm.at[pl.ds(base, TOK_W)], i2, sem)
    l3 = pltpu.async_copy(xpk_hbm.at[pl.ds(base, TOK_W)], xb, sem)
    l1.wait()
    l2.wait()
    l3.wait()
    c1 = pltpu.async_copy(xb, xs_hbm.at[i1], sem)
    c2 = pltpu.async_copy(xb, xs_hbm.at[i2], sem)
    c1.wait()
    c2.wait()


def _mlp_body(be_ref, nt_ref, xs_ref, wi_ref, wv_ref, wo_ref, ys_ref, acc):
    j = pl.program_id(0)
    i = pl.program_id(1)

    @pl.when(i < nt_ref[0])
    def _compute():
        xb = _unpack_bf16_pairs(xs_ref[...])
        a = jnp.dot(xb, wi_ref[0].astype(jnp.bfloat16),
                    preferred_element_type=jnp.float32)
        b = jnp.dot(xb, wv_ref[0].astype(jnp.bfloat16),
                    preferred_element_type=jnp.float32)
        c1 = 0.7978845608028654
        c2 = 0.7978845608028654 * 0.044715
        t = jnp.tanh(a * (c1 + c2 * (a * a)))
        half_a = 0.5 * a
        h = ((half_a + half_a * t) * b).astype(jnp.bfloat16)
        y = jnp.dot(h, wo_ref[0].astype(jnp.bfloat16),
                    preferred_element_type=jnp.float32)

        @pl.when(j == 0)
        def _set():
            acc[pl.ds(i * T, T), :] = y.astype(jnp.bfloat16)

        @pl.when(j == JF - 1)
        def _fin():
            ys_ref[...] = _pack_bf16_pairs(
                acc[pl.ds(i * T, T), :].astype(jnp.float32) + y)


def _combine_body(ys_hbm, s1_hbm, s2_hbm, w1_hbm, w2_hbm, out_hbm,
                  i1, i2, wv1, wv2, b1, b2, bout, sem):
    wid = lax.axis_index("s") * NC + lax.axis_index("c")
    base = wid * TOK_W
    l1 = pltpu.async_copy(s1_hbm.at[pl.ds(base, TOK_W)], i1, sem)
    l2 = pltpu.async_copy(s2_hbm.at[pl.ds(base, TOK_W)], i2, sem)
    l3 = pltpu.async_copy(w1_hbm.at[pl.ds(base, TOK_W)], wv1, sem)
    l4 = pltpu.async_copy(w2_hbm.at[pl.ds(base, TOK_W)], wv2, sem)
    l1.wait()
    l2.wait()
    l3.wait()
    l4.wait()
    c1 = pltpu.async_copy(ys_hbm.at[i1], b1, sem)
    c2 = pltpu.async_copy(ys_hbm.at[i2], b2, sem)
    c1.wait()
    c2.wait()

    def row(t, _):
        w1s = wv1[t]
        w2s = wv2[t]

        def col(c, _):
            v1 = b1[t, pl.ds(c * 16, 16)]
            v2 = b2[t, pl.ds(c * 16, 16)]
            lo1 = lax.bitcast_convert_type(v1 << 16, jnp.float32)
            lo2 = lax.bitcast_convert_type(v2 << 16, jnp.float32)
            hi1 = lax.bitcast_convert_type(v1 & jnp.int32(-65536), jnp.float32)
            hi2 = lax.bitcast_convert_type(v2 & jnp.int32(-65536), jnp.float32)
            bout[t, pl.ds(c * 16, 16)] = lo1 * w1s + lo2 * w2s
            bout[t, pl.ds(DP + c * 16, 16)] = hi1 * w1s + hi2 * w2s
            return 0
        return lax.fori_loop(0, DP // 16, col, 0, unroll=8)

    lax.fori_loop(0, TOK_W, row, 0)
    pltpu.sync_copy(bout, out_hbm.at[pl.ds(base, TOK_W)])


def kernel(hidden_states, gate_w, w_in, w_v, w_out):
    B = hidden_states.shape[0]
    x = hidden_states.reshape(S, D)

    logits, w1, w2, xpk, slot1, slot2, be, nt = pl.pallas_call(
        _router_body,
        grid=(SB,),
        in_specs=[
            pl.BlockSpec((RB, D), lambda i: (i, 0)),
            pl.BlockSpec((D, E), lambda i: (0, 0)),
        ],
        out_specs=[
            pl.BlockSpec((RB, E), lambda i: (i, 0)),
            pl.BlockSpec((RB, 16), lambda i: (i, 0)),
            pl.BlockSpec((RB, 16), lambda i: (i, 0)),
            pl.BlockSpec((RB, DP), lambda i: (i, 0)),
            pl.BlockSpec((S,), lambda i: (0,)),
            pl.BlockSpec((S,), lambda i: (0,)),
            pl.BlockSpec((NBLK,), lambda i: (0,)),
            pl.BlockSpec((1,), lambda i: (0,)),
        ],
        out_shape=(
            jax.ShapeDtypeStruct((S, E), jnp.float32),
            jax.ShapeDtypeStruct((S, 16), jnp.float32),
            jax.ShapeDtypeStruct((S, 16), jnp.float32),
            jax.ShapeDtypeStruct((S, DP), jnp.int32),
            jax.ShapeDtypeStruct((S,), jnp.int32),
            jax.ShapeDtypeStruct((S,), jnp.int32),
            jax.ShapeDtypeStruct((NBLK,), jnp.int32),
            jax.ShapeDtypeStruct((1,), jnp.int32),
        ),
        scratch_shapes=[
            pltpu.VMEM((1, E), jnp.float32),
            pltpu.VMEM((S, E), jnp.float32),
            pltpu.VMEM((S, 1), jnp.int32),
            pltpu.VMEM((S, 1), jnp.int32),
        ],
    )(x, gate_w)

    mesh = plsc.VectorSubcoreMesh(core_axis_name="c", subcore_axis_name="s")
    xs = pl.kernel(
        _dispatch_body,
        out_type=jax.ShapeDtypeStruct((NSLOT, DP), jnp.int32),
        mesh=mesh,
        scratch_types=[
            pltpu.VMEM((TOK_W, DP), jnp.int32),
            pltpu.VMEM((TOK_W,), jnp.int32),
            pltpu.VMEM((TOK_W,), jnp.int32),
            pltpu.SemaphoreType.DMA,
        ],
    )(xpk, slot1, slot2)

    ys = pl.pallas_call(
        _mlp_body,
        grid_spec=pltpu.PrefetchScalarGridSpec(
            num_scalar_prefetch=2,
            grid=(JF, NBLK),
            in_specs=[
                pl.BlockSpec(
                    (T, DP), lambda j, i, be, nt: (jnp.minimum(i, nt[0] - 1), 0)),
                pl.BlockSpec(
                    (1, D, Fb), lambda j, i, be, nt: (be[i], 0, j)),
                pl.BlockSpec(
                    (1, D, Fb), lambda j, i, be, nt: (be[i], 0, j)),
                pl.BlockSpec(
                    (1, Fb, D), lambda j, i, be, nt: (be[i], j, 0)),
            ],
            out_specs=pl.BlockSpec(
                (T, DP),
                lambda j, i, be, nt: (
                    jnp.where(j == 0, 0, jnp.minimum(i, nt[0] - 1)), 0)),
            scratch_shapes=[pltpu.VMEM((NSLOT, D), jnp.bfloat16)],
        ),
        out_shape=jax.ShapeDtypeStruct((NSLOT, DP), jnp.int32),
    )(be, nt, xs, w_in, w_v, w_out)

    out = pl.kernel(
        _combine_body,
        out_type=jax.ShapeDtypeStruct((S, D), jnp.float32),
        mesh=mesh,
        scratch_types=[
            pltpu.VMEM((TOK_W,), jnp.int32),
            pltpu.VMEM((TOK_W,), jnp.int32),
            pltpu.VMEM((TOK_W, 16), jnp.float32),
            pltpu.VMEM((TOK_W, 16), jnp.float32),
            pltpu.VMEM((TOK_W, DP), jnp.int32),
            pltpu.VMEM((TOK_W, DP), jnp.int32),
            pltpu.VMEM((TOK_W, D), jnp.float32),
            pltpu.SemaphoreType.DMA,
        ],
    )(ys, slot1, slot2, w1, w2)

    return out.reshape(B, S, D), logits.reshape(B, S, E)
